# baseline (speedup 1.0000x reference)
import functools

import jax
import jax.numpy as jnp
from jax import lax
from jax.experimental import pallas as pl
from jax.experimental.pallas import tpu as pltpu
from jax.experimental.pallas import tpu_sc as plsc

_N = 10000
_M = 5000
_E = 320000
_HID = 128
_NP = 10112
_RPT = _NP // 16
_CH = 128
_EPAD = 327680
_EPW = _EPAD // 32
_NCHUNK = _EPW // _CH
_MPAD = 5120
_NBLK = _NP // 128
_MBLK = _MPAD // 128

_f32 = jnp.float32


_sc_mesh = plsc.VectorSubcoreMesh(core_axis_name="c", subcore_axis_name="s")


_CHE = 64
_NCHUNKE = _EPW // _CHE


def _edge_body(src_hbm, dst_hbm, p_hbm, q_hbm, z_hbm,
               acc_out,
               idx_sA, idx_dA, pbufA, qbufA,
               idx_sB, idx_dB, pbufB, qbufB,
               semA, semB, semsA, semsB, acc_sh):
    cid = lax.axis_index("c")
    sid = lax.axis_index("s")
    wid = cid * 16 + sid
    r0 = sid * _RPT
    pltpu.sync_copy(z_hbm.at[pl.ds(r0, _RPT)], acc_sh.at[pl.ds(r0, _RPT)])
    plsc.subcore_barrier()
    e0 = wid * _EPW

    bufA = (idx_sA, idx_dA, pbufA, qbufA, semA, semsA)
    bufB = (idx_sB, idx_dB, pbufB, qbufB, semB, semsB)

    def load_and_gather(t, buf):
        ib, db, pb, qb, sem = buf[0], buf[1], buf[2], buf[3], buf[4]
        base = e0 + t * _CHE
        pltpu.sync_copy(src_hbm.at[pl.ds(base, _CHE)], ib)
        pltpu.sync_copy(dst_hbm.at[pl.ds(base, _CHE)], db)
        pltpu.async_copy(q_hbm.at[ib], qb, sem)
        pltpu.async_copy(p_hbm.at[db], pb, sem)

    def compute_rows(pb, qb, lo, hi):
        def row_body(i, c2):
            for j in range(8):
                sl = pl.ds(j * 16, 16)
                pb[i, sl] = jnp.maximum(pb[i, sl] + qb[i, sl], 0.0)
            return c2

        lax.fori_loop(lo, hi, row_body, 0)

    def phase(t, first, cur, nxt):
        ib, db, pb, qb, sem, sems = cur
        nsems = nxt[5]
        pltpu.make_async_copy(q_hbm.at[ib], qb, sem).wait()
        pltpu.make_async_copy(p_hbm.at[db], pb, sem).wait()
        compute_rows(pb, qb, 0, _CHE // 2)

        @pl.when(jnp.logical_not(first))
        def _():
            pltpu.make_async_copy(nxt[2], acc_sh.at[nxt[1]], nsems).wait()

        tn = t + 1

        @pl.when(tn < _NCHUNKE)
        def _():
            load_and_gather(tn, nxt)

        compute_rows(pb, qb, _CHE // 2, _CHE)
        pltpu.async_copy(pb, acc_sh.at[db], sems, add=True)

    load_and_gather(0, bufA)

    def pair_body(u, carry):
        phase(2 * u, u == 0, bufA, bufB)
        phase(2 * u + 1, jnp.asarray(False), bufB, bufA)
        return carry

    lax.fori_loop(0, _NCHUNKE // 2, pair_body, 0)
    pltpu.make_async_copy(pbufB, acc_sh.at[idx_dB], semsB).wait()
    plsc.subcore_barrier()
    pltpu.sync_copy(acc_sh.at[pl.ds(r0, _RPT)], acc_out.at[cid, pl.ds(r0, _RPT)])


_edge_sc = functools.partial(
    pl.kernel,
    out_type=[jax.ShapeDtypeStruct((2, _NP, _HID), _f32)],
    mesh=_sc_mesh,
    scratch_types=[
        pltpu.VMEM((_CHE,), jnp.int32),
        pltpu.VMEM((_CHE,), jnp.int32),
        pltpu.VMEM((_CHE, _HID), _f32),
        pltpu.VMEM((_CHE, _HID), _f32),
        pltpu.VMEM((_CHE,), jnp.int32),
        pltpu.VMEM((_CHE,), jnp.int32),
        pltpu.VMEM((_CHE, _HID), _f32),
        pltpu.VMEM((_CHE, _HID), _f32),
        pltpu.SemaphoreType.DMA,
        pltpu.SemaphoreType.DMA,
        pltpu.SemaphoreType.DMA,
        pltpu.SemaphoreType.DMA,
        pltpu.VMEM_SHARED((_NP, _HID), _f32),
    ],
)(_edge_body)


def _count_body(dst_hbm, z_hbm, ones_hbm,
                cnt_out,
                idx_d, ones_v, cnt_sh):
    cid = lax.axis_index("c")
    sid = lax.axis_index("s")
    wid = cid * 16 + sid
    r0 = sid * _RPT
    pltpu.sync_copy(z_hbm.at[pl.ds(r0, _RPT)], cnt_sh.at[pl.ds(r0, _RPT)])
    pltpu.sync_copy(ones_hbm, ones_v)
    plsc.subcore_barrier()
    e0 = wid * _EPW

    def chunk_body(t, carry):
        base = e0 + t * _CH
        pltpu.sync_copy(dst_hbm.at[pl.ds(base, _CH)], idx_d)
        pltpu.sync_copy(ones_v, cnt_sh.at[idx_d], add=True)
        return carry

    lax.fori_loop(0, _NCHUNK, chunk_body, 0)
    plsc.subcore_barrier()
    pltpu.sync_copy(cnt_sh.at[pl.ds(r0, _RPT)], cnt_out.at[cid, pl.ds(r0, _RPT)])


_count_sc = functools.partial(
    pl.kernel,
    out_type=[jax.ShapeDtypeStruct((2, _NP, _HID), _f32)],
    mesh=_sc_mesh,
    scratch_types=[
        pltpu.VMEM((_CH,), jnp.int32),
        pltpu.VMEM((_CH, _HID), _f32),
        pltpu.VMEM_SHARED((_NP, _HID), _f32),
    ],
)(_count_body)


def _pre_body(x_ref, wa_ref, ba_ref, p_ref, q_ref):
    wtop = wa_ref[0:_HID, :]
    wbot = wa_ref[_HID:2 * _HID, :]
    xb = x_ref[...]
    q_ref[...] = jnp.dot(xb, wbot, preferred_element_type=_f32)
    p_ref[...] = jnp.dot(xb, wtop - wbot, preferred_element_type=_f32) + ba_ref[...]


_pre_tc = pl.pallas_call(
    _pre_body,
    grid=(_NBLK,),
    in_specs=[pl.BlockSpec((128, _HID), lambda i: (i, 0)),
              pl.BlockSpec((2 * _HID, _HID), lambda i: (0, 0)),
              pl.BlockSpec((1, _HID), lambda i: (0, 0))],
    out_specs=[pl.BlockSpec((128, _HID), lambda i: (i, 0)),
               pl.BlockSpec((128, _HID), lambda i: (i, 0))],
    out_shape=[jax.ShapeDtypeStruct((_NP, _HID), _f32),
               jax.ShapeDtypeStruct((_NP, _HID), _f32)],
)


def _cnt_cols(cnt_ref):
    c0 = jnp.max(cnt_ref[0], axis=1, keepdims=True)
    c1 = jnp.max(cnt_ref[1], axis=1, keepdims=True)
    return c0 + c1


def _mid_body(acc_ref, cnt_ref, w1b_ref, b1b_ref, w2a_ref, b2a_ref,
              p2_ref, q2_ref):
    c = _cnt_cols(cnt_ref)
    mean = (acc_ref[0] + acc_ref[1]) / jnp.maximum(c, 1.0)
    h1 = (jnp.dot(mean, w1b_ref[...], preferred_element_type=_f32)
          + b1b_ref[...] * jnp.minimum(c, 1.0))
    wtop = w2a_ref[0:_HID, :]
    wbot = w2a_ref[_HID:2 * _HID, :]
    q2_ref[...] = jnp.dot(h1, wbot, preferred_element_type=_f32)
    p2_ref[...] = jnp.dot(h1, wtop - wbot, preferred_element_type=_f32) + b2a_ref[...]


_mid_tc = pl.pallas_call(
    _mid_body,
    grid=(_NBLK,),
    in_specs=[pl.BlockSpec((2, 128, _HID), lambda i: (0, i, 0)),
              pl.BlockSpec((2, 128, _HID), lambda i: (0, i, 0)),
              pl.BlockSpec((_HID, _HID), lambda i: (0, 0)),
              pl.BlockSpec((1, _HID), lambda i: (0, 0)),
              pl.BlockSpec((2 * _HID, _HID), lambda i: (0, 0)),
              pl.BlockSpec((1, _HID), lambda i: (0, 0))],
    out_specs=[pl.BlockSpec((128, _HID), lambda i: (i, 0)),
               pl.BlockSpec((128, _HID), lambda i: (i, 0))],
    out_shape=[jax.ShapeDtypeStruct((_NP, _HID), _f32),
               jax.ShapeDtypeStruct((_NP, _HID), _f32)],
)


def _head_body(acc_ref, cnt_ref, w2b_ref, b2b_ref, wh_ref, bh_ref,
               wbd_ref, bc_ref, o_ref):
    c = _cnt_cols(cnt_ref)
    mean = (acc_ref[0] + acc_ref[1]) / jnp.maximum(c, 1.0)
    hq = (jnp.dot(mean, w2b_ref[...], preferred_element_type=_f32)
          + b2b_ref[...] * jnp.minimum(c, 1.0))
    t = jnp.maximum(jnp.dot(hq, wh_ref[...], preferred_element_type=_f32)
                    + bh_ref[...], 0.0)
    z = jnp.dot(t, wbd_ref[...], preferred_element_type=_f32) + bc_ref[...]
    lanes = lax.broadcasted_iota(jnp.int32, (128, 128), 1)
    o_ref[...] = jnp.where((lanes >= 68) & (lanes < 79), jax.nn.sigmoid(z), z)


_head_tc = pl.pallas_call(
    _head_body,
    grid=(_MBLK,),
    in_specs=[pl.BlockSpec((2, 128, _HID), lambda i: (0, i, 0)),
              pl.BlockSpec((2, 128, _HID), lambda i: (0, i, 0)),
              pl.BlockSpec((_HID, _HID), lambda i: (0, 0)),
              pl.BlockSpec((1, _HID), lambda i: (0, 0)),
              pl.BlockSpec((_HID, 4 * _HID), lambda i: (0, 0)),
              pl.BlockSpec((1, 4 * _HID), lambda i: (0, 0)),
              pl.BlockSpec((4 * _HID, 128), lambda i: (0, 0)),
              pl.BlockSpec((1, 128), lambda i: (0, 0))],
    out_specs=pl.BlockSpec((128, 128), lambda i: (i, 0)),
    out_shape=jax.ShapeDtypeStruct((_MPAD, 128), _f32),
)


def kernel(x, edge_index, W1a, b1a, W1b, b1b, W2a, b2a, W2b, b2b,
           Wf1, bf1, Wf2, bf2, Ws1, bs1, Ws2, bs2,
           Wo1, bo1, Wo2, bo2, Wm1, bm1, Wm2, bm2):
    src = edge_index[0].astype(jnp.int32)
    dst = edge_index[1].astype(jnp.int32)
    pad_e = jnp.full((_EPAD - _E,), _N, jnp.int32)
    src_p = jnp.concatenate([src, pad_e])
    dst_p = jnp.concatenate([dst, pad_e])
    xp = jnp.pad(x.astype(_f32), ((0, _NP - _N), (0, 0)))
    z_nd = jnp.zeros((_NP, _HID), _f32)
    ones_ch = jnp.ones((_CH, _HID), _f32)

    (cnt,) = _count_sc(dst_p, z_nd, ones_ch)
    p1, q1 = _pre_tc(xp, W1a, b1a.reshape(1, -1))
    (acc1,) = _edge_sc(src_p, dst_p, p1, q1, z_nd)
    p2, q2 = _mid_tc(acc1, cnt, W1b, b1b.reshape(1, -1), W2a, b2a.reshape(1, -1))
    (acc2,) = _edge_sc(src_p, dst_p, p2, q2, z_nd)

    wh = jnp.concatenate([Wf1, Ws1, Wo1, Wm1], axis=1)
    bh = jnp.concatenate([bf1, bs1, bo1, bm1]).reshape(1, -1)
    wbd = jnp.zeros((4 * _HID, 128), _f32)
    wbd = wbd.at[0:128, 0:32].set(Wf2)
    wbd = wbd.at[128:256, 32:38].set(Ws2)
    wbd = wbd.at[256:384, 38:68].set(Wo2)
    wbd = wbd.at[384:512, 68:79].set(Wm2)
    bc = jnp.zeros((128,), _f32)
    bc = bc.at[0:32].set(bf2).at[32:38].set(bs2).at[38:68].set(bo2).at[68:79].set(bm2)

    out = _head_tc(acc2, cnt, W2b, b2b.reshape(1, -1), wh, bh, wbd,
                   bc.reshape(1, -1))
    return out[:_M, :79]


# speedup vs baseline: 1.0111x; 1.0111x over previous
import functools

import jax
import jax.numpy as jnp
from jax import lax
from jax.experimental import pallas as pl
from jax.experimental.pallas import tpu as pltpu
from jax.experimental.pallas import tpu_sc as plsc

_N = 10000
_M = 5000
_E = 320000
_HID = 128
_NP = 10112
_RPT = _NP // 16
_CH = 128
_EPAD = 327680
_EPW = _EPAD // 32
_NCHUNK = _EPW // _CH
_MPAD = 5120
_NBLK = _NP // 128
_MBLK = _MPAD // 128

_f32 = jnp.float32


_sc_mesh = plsc.VectorSubcoreMesh(core_axis_name="c", subcore_axis_name="s")


def _edge_body(src_hbm, dst_hbm, p_hbm, q_hbm, z_hbm,
               acc_out,
               idx_s, idx_d, pbuf, qbuf, sem1, sem2, acc_sh):
    cid = lax.axis_index("c")
    sid = lax.axis_index("s")
    wid = cid * 16 + sid
    r0 = sid * _RPT
    pltpu.sync_copy(z_hbm.at[pl.ds(r0, _RPT)], acc_sh.at[pl.ds(r0, _RPT)])
    plsc.subcore_barrier()
    e0 = wid * _EPW

    def chunk_body(t, carry):
        base = e0 + t * _CH
        pltpu.sync_copy(src_hbm.at[pl.ds(base, _CH)], idx_s)
        pltpu.sync_copy(dst_hbm.at[pl.ds(base, _CH)], idx_d)
        cq = pltpu.async_copy(q_hbm.at[idx_s], qbuf, sem1)
        cp = pltpu.async_copy(p_hbm.at[idx_d], pbuf, sem2)
        cq.wait()
        cp.wait()

        @plsc.parallel_loop(0, _CH, unroll=4)
        def row_body(i):
            for j in range(8):
                sl = pl.ds(j * 16, 16)
                pbuf[i, sl] = jnp.maximum(pbuf[i, sl] + qbuf[i, sl], 0.0)

        pltpu.sync_copy(pbuf, acc_sh.at[idx_d], add=True)
        return carry

    lax.fori_loop(0, _NCHUNK, chunk_body, 0)
    plsc.subcore_barrier()
    pltpu.sync_copy(acc_sh.at[pl.ds(r0, _RPT)], acc_out.at[cid, pl.ds(r0, _RPT)])


_edge_sc = functools.partial(
    pl.kernel,
    out_type=[jax.ShapeDtypeStruct((2, _NP, _HID), _f32)],
    mesh=_sc_mesh,
    scratch_types=[
        pltpu.VMEM((_CH,), jnp.int32),
        pltpu.VMEM((_CH,), jnp.int32),
        pltpu.VMEM((_CH, _HID), _f32),
        pltpu.VMEM((_CH, _HID), _f32),
        pltpu.SemaphoreType.DMA,
        pltpu.SemaphoreType.DMA,
        pltpu.VMEM_SHARED((_NP, _HID), _f32),
    ],
)(_edge_body)


def _count_body(dst_hbm, z_hbm, ones_hbm,
                cnt_out,
                idx_d, ones_v, cnt_sh):
    cid = lax.axis_index("c")
    sid = lax.axis_index("s")
    wid = cid * 16 + sid
    r0 = sid * _RPT
    pltpu.sync_copy(z_hbm.at[pl.ds(r0, _RPT)], cnt_sh.at[pl.ds(r0, _RPT)])
    pltpu.sync_copy(ones_hbm, ones_v)
    plsc.subcore_barrier()
    e0 = wid * _EPW

    def chunk_body(t, carry):
        base = e0 + t * _CH
        pltpu.sync_copy(dst_hbm.at[pl.ds(base, _CH)], idx_d)
        pltpu.sync_copy(ones_v, cnt_sh.at[idx_d], add=True)
        return carry

    lax.fori_loop(0, _NCHUNK, chunk_body, 0)
    plsc.subcore_barrier()
    pltpu.sync_copy(cnt_sh.at[pl.ds(r0, _RPT)], cnt_out.at[cid, pl.ds(r0, _RPT)])


_count_sc = functools.partial(
    pl.kernel,
    out_type=[jax.ShapeDtypeStruct((2, _NP, _HID), _f32)],
    mesh=_sc_mesh,
    scratch_types=[
        pltpu.VMEM((_CH,), jnp.int32),
        pltpu.VMEM((_CH, _HID), _f32),
        pltpu.VMEM_SHARED((_NP, _HID), _f32),
    ],
)(_count_body)


def _pre_body(x_ref, wa_ref, ba_ref, p_ref, q_ref):
    wtop = wa_ref[0:_HID, :]
    wbot = wa_ref[_HID:2 * _HID, :]
    xb = x_ref[...]
    q_ref[...] = jnp.dot(xb, wbot, preferred_element_type=_f32)
    p_ref[...] = jnp.dot(xb, wtop - wbot, preferred_element_type=_f32) + ba_ref[...]


_pre_tc = pl.pallas_call(
    _pre_body,
    grid=(_NBLK,),
    in_specs=[pl.BlockSpec((128, _HID), lambda i: (i, 0)),
              pl.BlockSpec((2 * _HID, _HID), lambda i: (0, 0)),
              pl.BlockSpec((1, _HID), lambda i: (0, 0))],
    out_specs=[pl.BlockSpec((128, _HID), lambda i: (i, 0)),
               pl.BlockSpec((128, _HID), lambda i: (i, 0))],
    out_shape=[jax.ShapeDtypeStruct((_NP, _HID), _f32),
               jax.ShapeDtypeStruct((_NP, _HID), _f32)],
)


def _cnt_cols(cnt_ref):
    c0 = jnp.max(cnt_ref[0], axis=1, keepdims=True)
    c1 = jnp.max(cnt_ref[1], axis=1, keepdims=True)
    return c0 + c1


def _mid_body(acc_ref, cnt_ref, w1b_ref, b1b_ref, w2a_ref, b2a_ref,
              p2_ref, q2_ref):
    c = _cnt_cols(cnt_ref)
    mean = (acc_ref[0] + acc_ref[1]) / jnp.maximum(c, 1.0)
    h1 = (jnp.dot(mean, w1b_ref[...], preferred_element_type=_f32)
          + b1b_ref[...] * jnp.minimum(c, 1.0))
    wtop = w2a_ref[0:_HID, :]
    wbot = w2a_ref[_HID:2 * _HID, :]
    q2_ref[...] = jnp.dot(h1, wbot, preferred_element_type=_f32)
    p2_ref[...] = jnp.dot(h1, wtop - wbot, preferred_element_type=_f32) + b2a_ref[...]


_mid_tc = pl.pallas_call(
    _mid_body,
    grid=(_NBLK,),
    in_specs=[pl.BlockSpec((2, 128, _HID), lambda i: (0, i, 0)),
              pl.BlockSpec((2, 128, _HID), lambda i: (0, i, 0)),
              pl.BlockSpec((_HID, _HID), lambda i: (0, 0)),
              pl.BlockSpec((1, _HID), lambda i: (0, 0)),
              pl.BlockSpec((2 * _HID, _HID), lambda i: (0, 0)),
              pl.BlockSpec((1, _HID), lambda i: (0, 0))],
    out_specs=[pl.BlockSpec((128, _HID), lambda i: (i, 0)),
               pl.BlockSpec((128, _HID), lambda i: (i, 0))],
    out_shape=[jax.ShapeDtypeStruct((_NP, _HID), _f32),
               jax.ShapeDtypeStruct((_NP, _HID), _f32)],
)


def _head_body(acc_ref, cnt_ref, w2b_ref, b2b_ref, wh_ref, bh_ref,
               wbd_ref, bc_ref, o_ref):
    c = _cnt_cols(cnt_ref)
    mean = (acc_ref[0] + acc_ref[1]) / jnp.maximum(c, 1.0)
    hq = (jnp.dot(mean, w2b_ref[...], preferred_element_type=_f32)
          + b2b_ref[...] * jnp.minimum(c, 1.0))
    t = jnp.maximum(jnp.dot(hq, wh_ref[...], preferred_element_type=_f32)
                    + bh_ref[...], 0.0)
    z = jnp.dot(t, wbd_ref[...], preferred_element_type=_f32) + bc_ref[...]
    lanes = lax.broadcasted_iota(jnp.int32, (128, 128), 1)
    o_ref[...] = jnp.where((lanes >= 68) & (lanes < 79), jax.nn.sigmoid(z), z)


_head_tc = pl.pallas_call(
    _head_body,
    grid=(_MBLK,),
    in_specs=[pl.BlockSpec((2, 128, _HID), lambda i: (0, i, 0)),
              pl.BlockSpec((2, 128, _HID), lambda i: (0, i, 0)),
              pl.BlockSpec((_HID, _HID), lambda i: (0, 0)),
              pl.BlockSpec((1, _HID), lambda i: (0, 0)),
              pl.BlockSpec((_HID, 4 * _HID), lambda i: (0, 0)),
              pl.BlockSpec((1, 4 * _HID), lambda i: (0, 0)),
              pl.BlockSpec((4 * _HID, 128), lambda i: (0, 0)),
              pl.BlockSpec((1, 128), lambda i: (0, 0))],
    out_specs=pl.BlockSpec((128, 128), lambda i: (i, 0)),
    out_shape=jax.ShapeDtypeStruct((_MPAD, 128), _f32),
)


def kernel(x, edge_index, W1a, b1a, W1b, b1b, W2a, b2a, W2b, b2b,
           Wf1, bf1, Wf2, bf2, Ws1, bs1, Ws2, bs2,
           Wo1, bo1, Wo2, bo2, Wm1, bm1, Wm2, bm2):
    src = edge_index[0].astype(jnp.int32)
    dst = edge_index[1].astype(jnp.int32)
    pad_e = jnp.full((_EPAD - _E,), _N, jnp.int32)
    src_p = jnp.concatenate([src, pad_e])
    dst_p = jnp.concatenate([dst, pad_e])
    xp = jnp.pad(x.astype(_f32), ((0, _NP - _N), (0, 0)))
    z_nd = jnp.zeros((_NP, _HID), _f32)
    ones_ch = jnp.ones((_CH, _HID), _f32)

    (cnt,) = _count_sc(dst_p, z_nd, ones_ch)
    p1, q1 = _pre_tc(xp, W1a, b1a.reshape(1, -1))
    (acc1,) = _edge_sc(src_p, dst_p, p1, q1, z_nd)
    p2, q2 = _mid_tc(acc1, cnt, W1b, b1b.reshape(1, -1), W2a, b2a.reshape(1, -1))
    (acc2,) = _edge_sc(src_p, dst_p, p2, q2, z_nd)

    wh = jnp.concatenate([Wf1, Ws1, Wo1, Wm1], axis=1)
    bh = jnp.concatenate([bf1, bs1, bo1, bm1]).reshape(1, -1)
    wbd = jnp.zeros((4 * _HID, 128), _f32)
    wbd = wbd.at[0:128, 0:32].set(Wf2)
    wbd = wbd.at[128:256, 32:38].set(Ws2)
    wbd = wbd.at[256:384, 38:68].set(Wo2)
    wbd = wbd.at[384:512, 68:79].set(Wm2)
    bc = jnp.zeros((128,), _f32)
    bc = bc.at[0:32].set(bf2).at[32:38].set(bs2).at[38:68].set(bo2).at[68:79].set(bm2)

    out = _head_tc(acc2, cnt, W2b, b2b.reshape(1, -1), wh, bh, wbd,
                   bc.reshape(1, -1))
    return out[:_M, :79]


# speedup vs baseline: 1.0115x; 1.0004x over previous
import functools

import jax
import jax.numpy as jnp
from jax import lax
from jax.experimental import pallas as pl
from jax.experimental.pallas import tpu as pltpu
from jax.experimental.pallas import tpu_sc as plsc

_N = 10000
_M = 5000
_E = 320000
_HID = 128
_NP = 10112
_RPT = _NP // 16
_CH = 128
_EPAD = 327680
_EPW = _EPAD // 32
_NCHUNK = _EPW // _CH
_MPAD = 5120
_NBLK = _NP // 128
_MBLK = _MPAD // 128

_f32 = jnp.float32


_sc_mesh = plsc.VectorSubcoreMesh(core_axis_name="c", subcore_axis_name="s")


def _edge_body(src_hbm, dst_hbm, p_hbm, q_hbm, z_hbm,
               acc_out,
               idx_s, idx_d, pbuf, qbuf, sem1, sem2, acc_sh):
    cid = lax.axis_index("c")
    sid = lax.axis_index("s")
    wid = cid * 16 + sid
    r0 = sid * _RPT
    pltpu.sync_copy(z_hbm.at[pl.ds(r0, _RPT)], acc_sh.at[pl.ds(r0, _RPT)])
    plsc.subcore_barrier()
    e0 = wid * _EPW

    def chunk_body(t, carry):
        base = e0 + t * _CH
        pltpu.sync_copy(src_hbm.at[pl.ds(base, _CH)], idx_s)
        pltpu.sync_copy(dst_hbm.at[pl.ds(base, _CH)], idx_d)
        cq = pltpu.async_copy(q_hbm.at[idx_s], qbuf, sem1)
        cp = pltpu.async_copy(p_hbm.at[idx_d], pbuf, sem2)
        cq.wait()
        cp.wait()

        def row_body(i, c2):
            for j in range(8):
                sl = pl.ds(j * 16, 16)
                pbuf[i, sl] = jnp.maximum(pbuf[i, sl] + qbuf[i, sl], 0.0)
            return c2

        lax.fori_loop(0, _CH, row_body, 0)

        pltpu.sync_copy(pbuf, acc_sh.at[idx_d], add=True)
        return carry

    lax.fori_loop(0, _NCHUNK, chunk_body, 0)
    plsc.subcore_barrier()
    pltpu.sync_copy(acc_sh.at[pl.ds(r0, _RPT)], acc_out.at[cid, pl.ds(r0, _RPT)])


_edge_sc = functools.partial(
    pl.kernel,
    out_type=[jax.ShapeDtypeStruct((2, _NP, _HID), _f32)],
    mesh=_sc_mesh,
    scratch_types=[
        pltpu.VMEM((_CH,), jnp.int32),
        pltpu.VMEM((_CH,), jnp.int32),
        pltpu.VMEM((_CH, _HID), _f32),
        pltpu.VMEM((_CH, _HID), _f32),
        pltpu.SemaphoreType.DMA,
        pltpu.SemaphoreType.DMA,
        pltpu.VMEM_SHARED((_NP, _HID), _f32),
    ],
)(_edge_body)


def _count_body(dst_hbm, z_hbm, ones_hbm,
                cnt_out,
                idx_d, ones_v, cnt_sh):
    cid = lax.axis_index("c")
    sid = lax.axis_index("s")
    wid = cid * 16 + sid
    r0 = sid * _RPT
    pltpu.sync_copy(z_hbm.at[pl.ds(r0, _RPT)], cnt_sh.at[pl.ds(r0, _RPT)])
    pltpu.sync_copy(ones_hbm, ones_v)
    plsc.subcore_barrier()
    e0 = wid * _EPW

    def chunk_body(t, carry):
        base = e0 + t * _CH
        pltpu.sync_copy(dst_hbm.at[pl.ds(base, _CH)], idx_d)
        pltpu.sync_copy(ones_v, cnt_sh.at[idx_d], add=True)
        return carry

    lax.fori_loop(0, _NCHUNK, chunk_body, 0)
    plsc.subcore_barrier()
    pltpu.sync_copy(cnt_sh.at[pl.ds(r0, _RPT)], cnt_out.at[cid, pl.ds(r0, _RPT)])


_count_sc = functools.partial(
    pl.kernel,
    out_type=[jax.ShapeDtypeStruct((2, _NP, _HID), _f32)],
    mesh=_sc_mesh,
    scratch_types=[
        pltpu.VMEM((_CH,), jnp.int32),
        pltpu.VMEM((_CH, _HID), _f32),
        pltpu.VMEM_SHARED((_NP, _HID), _f32),
    ],
)(_count_body)


def _pre_body(x_ref, wa_ref, ba_ref, p_ref, q_ref):
    wtop = wa_ref[0:_HID, :]
    wbot = wa_ref[_HID:2 * _HID, :]
    xb = x_ref[...]
    q_ref[...] = jnp.dot(xb, wbot, preferred_element_type=_f32)
    p_ref[...] = jnp.dot(xb, wtop - wbot, preferred_element_type=_f32) + ba_ref[...]


_pre_tc = pl.pallas_call(
    _pre_body,
    grid=(_NBLK,),
    in_specs=[pl.BlockSpec((128, _HID), lambda i: (i, 0)),
              pl.BlockSpec((2 * _HID, _HID), lambda i: (0, 0)),
              pl.BlockSpec((1, _HID), lambda i: (0, 0))],
    out_specs=[pl.BlockSpec((128, _HID), lambda i: (i, 0)),
               pl.BlockSpec((128, _HID), lambda i: (i, 0))],
    out_shape=[jax.ShapeDtypeStruct((_NP, _HID), _f32),
               jax.ShapeDtypeStruct((_NP, _HID), _f32)],
)


def _cnt_cols(cnt_ref):
    c0 = jnp.max(cnt_ref[0], axis=1, keepdims=True)
    c1 = jnp.max(cnt_ref[1], axis=1, keepdims=True)
    return c0 + c1


def _mid_body(acc_ref, cnt_ref, w1b_ref, b1b_ref, w2a_ref, b2a_ref,
              p2_ref, q2_ref):
    c = _cnt_cols(cnt_ref)
    mean = (acc_ref[0] + acc_ref[1]) / jnp.maximum(c, 1.0)
    h1 = (jnp.dot(mean, w1b_ref[...], preferred_element_type=_f32)
          + b1b_ref[...] * jnp.minimum(c, 1.0))
    wtop = w2a_ref[0:_HID, :]
    wbot = w2a_ref[_HID:2 * _HID, :]
    q2_ref[...] = jnp.dot(h1, wbot, preferred_element_type=_f32)
    p2_ref[...] = jnp.dot(h1, wtop - wbot, preferred_element_type=_f32) + b2a_ref[...]


_mid_tc = pl.pallas_call(
    _mid_body,
    grid=(_NBLK,),
    in_specs=[pl.BlockSpec((2, 128, _HID), lambda i: (0, i, 0)),
              pl.BlockSpec((2, 128, _HID), lambda i: (0, i, 0)),
              pl.BlockSpec((_HID, _HID), lambda i: (0, 0)),
              pl.BlockSpec((1, _HID), lambda i: (0, 0)),
              pl.BlockSpec((2 * _HID, _HID), lambda i: (0, 0)),
              pl.BlockSpec((1, _HID), lambda i: (0, 0))],
    out_specs=[pl.BlockSpec((128, _HID), lambda i: (i, 0)),
               pl.BlockSpec((128, _HID), lambda i: (i, 0))],
    out_shape=[jax.ShapeDtypeStruct((_NP, _HID), _f32),
               jax.ShapeDtypeStruct((_NP, _HID), _f32)],
)


def _head_body(acc_ref, cnt_ref, w2b_ref, b2b_ref, wh_ref, bh_ref,
               wbd_ref, bc_ref, o_ref):
    c = _cnt_cols(cnt_ref)
    mean = (acc_ref[0] + acc_ref[1]) / jnp.maximum(c, 1.0)
    hq = (jnp.dot(mean, w2b_ref[...], preferred_element_type=_f32)
          + b2b_ref[...] * jnp.minimum(c, 1.0))
    t = jnp.maximum(jnp.dot(hq, wh_ref[...], preferred_element_type=_f32)
                    + bh_ref[...], 0.0)
    z = jnp.dot(t, wbd_ref[...], preferred_element_type=_f32) + bc_ref[...]
    lanes = lax.broadcasted_iota(jnp.int32, (128, 128), 1)
    o_ref[...] = jnp.where((lanes >= 68) & (lanes < 79), jax.nn.sigmoid(z), z)


_head_tc = pl.pallas_call(
    _head_body,
    grid=(_MBLK,),
    in_specs=[pl.BlockSpec((2, 128, _HID), lambda i: (0, i, 0)),
              pl.BlockSpec((2, 128, _HID), lambda i: (0, i, 0)),
              pl.BlockSpec((_HID, _HID), lambda i: (0, 0)),
              pl.BlockSpec((1, _HID), lambda i: (0, 0)),
              pl.BlockSpec((_HID, 4 * _HID), lambda i: (0, 0)),
              pl.BlockSpec((1, 4 * _HID), lambda i: (0, 0)),
              pl.BlockSpec((4 * _HID, 128), lambda i: (0, 0)),
              pl.BlockSpec((1, 128), lambda i: (0, 0))],
    out_specs=pl.BlockSpec((128, 128), lambda i: (i, 0)),
    out_shape=jax.ShapeDtypeStruct((_MPAD, 128), _f32),
)


def kernel(x, edge_index, W1a, b1a, W1b, b1b, W2a, b2a, W2b, b2b,
           Wf1, bf1, Wf2, bf2, Ws1, bs1, Ws2, bs2,
           Wo1, bo1, Wo2, bo2, Wm1, bm1, Wm2, bm2):
    src = edge_index[0].astype(jnp.int32)
    dst = edge_index[1].astype(jnp.int32)
    pad_e = jnp.full((_EPAD - _E,), _N, jnp.int32)
    src_p = jnp.concatenate([src, pad_e])
    dst_p = jnp.concatenate([dst, pad_e])
    xp = jnp.pad(x.astype(_f32), ((0, _NP - _N), (0, 0)))
    z_nd = jnp.zeros((_NP, _HID), _f32)
    ones_ch = jnp.ones((_CH, _HID), _f32)

    (cnt,) = _count_sc(dst_p, z_nd, ones_ch)
    p1, q1 = _pre_tc(xp, W1a, b1a.reshape(1, -1))
    (acc1,) = _edge_sc(src_p, dst_p, p1, q1, z_nd)
    p2, q2 = _mid_tc(acc1, cnt, W1b, b1b.reshape(1, -1), W2a, b2a.reshape(1, -1))
    (acc2,) = _edge_sc(src_p, dst_p, p2, q2, z_nd)

    wh = jnp.concatenate([Wf1, Ws1, Wo1, Wm1], axis=1)
    bh = jnp.concatenate([bf1, bs1, bo1, bm1]).reshape(1, -1)
    wbd = jnp.zeros((4 * _HID, 128), _f32)
    wbd = wbd.at[0:128, 0:32].set(Wf2)
    wbd = wbd.at[128:256, 32:38].set(Ws2)
    wbd = wbd.at[256:384, 38:68].set(Wo2)
    wbd = wbd.at[384:512, 68:79].set(Wm2)
    bc = jnp.zeros((128,), _f32)
    bc = bc.at[0:32].set(bf2).at[32:38].set(bs2).at[38:68].set(bo2).at[68:79].set(bm2)

    out = _head_tc(acc2, cnt, W2b, b2b.reshape(1, -1), wh, bh, wbd,
                   bc.reshape(1, -1))
    return out[:_M, :79]


# speedup vs baseline: 1.7813x; 1.7611x over previous
import functools

import jax
import jax.numpy as jnp
from jax import lax
from jax.experimental import pallas as pl
from jax.experimental.pallas import tpu as pltpu
from jax.experimental.pallas import tpu_sc as plsc

_N = 10000
_M = 5000
_E = 320000
_HID = 128
_NP = 10112
_RPT = _NP // 16
_CH = 128
_EPAD = 327680
_EPW = _EPAD // 32
_NCHUNK = _EPW // _CH
_MPAD = 5120
_NBLK = _NP // 128
_MBLK = _MPAD // 128

_f32 = jnp.float32


_sc_mesh = plsc.VectorSubcoreMesh(core_axis_name="c", subcore_axis_name="s")


def _edge_body(src_hbm, dst_hbm, p_hbm, q_hbm, z_hbm,
               acc_out,
               idx_s, idx_d, pbuf, qbuf, sem1, sem2, acc_sh):
    cid = lax.axis_index("c")
    sid = lax.axis_index("s")
    wid = cid * 16 + sid
    r0 = sid * _RPT
    pltpu.sync_copy(z_hbm.at[pl.ds(r0, _RPT)], acc_sh.at[pl.ds(r0, _RPT)])
    plsc.subcore_barrier()
    e0 = wid * _EPW

    def chunk_body(t, carry):
        base = e0 + t * _CH
        pltpu.sync_copy(src_hbm.at[pl.ds(base, _CH)], idx_s)
        pltpu.sync_copy(dst_hbm.at[pl.ds(base, _CH)], idx_d)
        cq = pltpu.async_copy(q_hbm.at[idx_s], qbuf, sem1)
        cp = pltpu.async_copy(p_hbm.at[idx_d], pbuf, sem2)
        cq.wait()
        cp.wait()

        def row_body(i, c2):
            for j in range(8):
                sl = pl.ds(j * 16, 16)
                pbuf[i, sl] = jnp.maximum(pbuf[i, sl] + qbuf[i, sl], 0.0)
            return c2

        lax.fori_loop(0, _CH, row_body, 0)

        pltpu.sync_copy(pbuf, acc_sh.at[idx_d], add=True)
        return carry

    lax.fori_loop(0, _NCHUNK, chunk_body, 0)
    plsc.subcore_barrier()
    pltpu.sync_copy(acc_sh.at[pl.ds(r0, _RPT)], acc_out.at[cid, pl.ds(r0, _RPT)])


_edge_sc = functools.partial(
    pl.kernel,
    out_type=[jax.ShapeDtypeStruct((2, _NP, _HID), _f32)],
    mesh=_sc_mesh,
    scratch_types=[
        pltpu.VMEM((_CH,), jnp.int32),
        pltpu.VMEM((_CH,), jnp.int32),
        pltpu.VMEM((_CH, _HID), _f32),
        pltpu.VMEM((_CH, _HID), _f32),
        pltpu.SemaphoreType.DMA,
        pltpu.SemaphoreType.DMA,
        pltpu.VMEM_SHARED((_NP, _HID), _f32),
    ],
)(_edge_body)


def _count_body(dst_hbm, z_hbm, ones_hbm,
                cnt_out,
                idx_d, ones_v, cnt_sh):
    cid = lax.axis_index("c")
    sid = lax.axis_index("s")
    wid = cid * 16 + sid
    r0 = sid * _RPT
    pltpu.sync_copy(z_hbm.at[pl.ds(r0, _RPT)], cnt_sh.at[pl.ds(r0, _RPT)])
    pltpu.sync_copy(ones_hbm, ones_v)
    plsc.subcore_barrier()
    e0 = wid * _EPW

    def chunk_body(t, carry):
        base = e0 + t * _CH
        pltpu.sync_copy(dst_hbm.at[pl.ds(base, _CH)], idx_d)
        pltpu.sync_copy(ones_v, cnt_sh.at[idx_d], add=True)
        return carry

    lax.fori_loop(0, _NCHUNK, chunk_body, 0)
    plsc.subcore_barrier()
    pltpu.sync_copy(cnt_sh.at[pl.ds(r0, _RPT)], cnt_out.at[cid, pl.ds(r0, _RPT)])


_count_sc = functools.partial(
    pl.kernel,
    out_type=[jax.ShapeDtypeStruct((2, _NP, _HID), _f32)],
    mesh=_sc_mesh,
    scratch_types=[
        pltpu.VMEM((_CH,), jnp.int32),
        pltpu.VMEM((_CH, _HID), _f32),
        pltpu.VMEM_SHARED((_NP, _HID), _f32),
    ],
)(_count_body)


def _pre_body(x_ref, wa_ref, ba_ref, p_ref, q_ref):
    wtop = wa_ref[0:_HID, :]
    wbot = wa_ref[_HID:2 * _HID, :]
    xb = x_ref[...]
    q_ref[...] = jnp.dot(xb, wbot, preferred_element_type=_f32)
    p_ref[...] = jnp.dot(xb, wtop - wbot, preferred_element_type=_f32) + ba_ref[...]


_pre_tc = pl.pallas_call(
    _pre_body,
    grid=(_NBLK,),
    in_specs=[pl.BlockSpec((128, _HID), lambda i: (i, 0)),
              pl.BlockSpec((2 * _HID, _HID), lambda i: (0, 0)),
              pl.BlockSpec((1, _HID), lambda i: (0, 0))],
    out_specs=[pl.BlockSpec((128, _HID), lambda i: (i, 0)),
               pl.BlockSpec((128, _HID), lambda i: (i, 0))],
    out_shape=[jax.ShapeDtypeStruct((_NP, _HID), _f32),
               jax.ShapeDtypeStruct((_NP, _HID), _f32)],
)


def _cnt_cols(cnt_ref):
    c0 = jnp.max(cnt_ref[0], axis=1, keepdims=True)
    c1 = jnp.max(cnt_ref[1], axis=1, keepdims=True)
    return c0 + c1


def _mid_body(acc_ref, cnt_ref, w1b_ref, b1b_ref, w2a_ref, b2a_ref,
              p2_ref, q2_ref):
    c = _cnt_cols(cnt_ref)
    mean = (acc_ref[0] + acc_ref[1]) / jnp.maximum(c, 1.0)
    h1 = (jnp.dot(mean, w1b_ref[...], preferred_element_type=_f32)
          + b1b_ref[...] * jnp.minimum(c, 1.0))
    wtop = w2a_ref[0:_HID, :]
    wbot = w2a_ref[_HID:2 * _HID, :]
    q2_ref[...] = jnp.dot(h1, wbot, preferred_element_type=_f32)
    p2_ref[...] = jnp.dot(h1, wtop - wbot, preferred_element_type=_f32) + b2a_ref[...]


_mid_tc = pl.pallas_call(
    _mid_body,
    grid=(_NBLK,),
    in_specs=[pl.BlockSpec((2, 128, _HID), lambda i: (0, i, 0)),
              pl.BlockSpec((2, 128, _HID), lambda i: (0, i, 0)),
              pl.BlockSpec((_HID, _HID), lambda i: (0, 0)),
              pl.BlockSpec((1, _HID), lambda i: (0, 0)),
              pl.BlockSpec((2 * _HID, _HID), lambda i: (0, 0)),
              pl.BlockSpec((1, _HID), lambda i: (0, 0))],
    out_specs=[pl.BlockSpec((128, _HID), lambda i: (i, 0)),
               pl.BlockSpec((128, _HID), lambda i: (i, 0))],
    out_shape=[jax.ShapeDtypeStruct((_NP, _HID), _f32),
               jax.ShapeDtypeStruct((_NP, _HID), _f32)],
)


def _head_body(acc_ref, cnt_ref, w2b_ref, b2b_ref, wh_ref, bh_ref,
               wbd_ref, bc_ref, o_ref):
    c = _cnt_cols(cnt_ref)
    mean = (acc_ref[0] + acc_ref[1]) / jnp.maximum(c, 1.0)
    hq = (jnp.dot(mean, w2b_ref[...], preferred_element_type=_f32)
          + b2b_ref[...] * jnp.minimum(c, 1.0))
    t = jnp.maximum(jnp.dot(hq, wh_ref[...], preferred_element_type=_f32)
                    + bh_ref[...], 0.0)
    z = jnp.dot(t, wbd_ref[...], preferred_element_type=_f32) + bc_ref[...]
    lanes = lax.broadcasted_iota(jnp.int32, (128, 128), 1)
    o_ref[...] = jnp.where((lanes >= 68) & (lanes < 79), jax.nn.sigmoid(z), z)


_head_tc = pl.pallas_call(
    _head_body,
    grid=(_MBLK,),
    in_specs=[pl.BlockSpec((2, 128, _HID), lambda i: (0, i, 0)),
              pl.BlockSpec((2, 128, _HID), lambda i: (0, i, 0)),
              pl.BlockSpec((_HID, _HID), lambda i: (0, 0)),
              pl.BlockSpec((1, _HID), lambda i: (0, 0)),
              pl.BlockSpec((_HID, 4 * _HID), lambda i: (0, 0)),
              pl.BlockSpec((1, 4 * _HID), lambda i: (0, 0)),
              pl.BlockSpec((4 * _HID, 128), lambda i: (0, 0)),
              pl.BlockSpec((1, 128), lambda i: (0, 0))],
    out_specs=pl.BlockSpec((128, 128), lambda i: (i, 0)),
    out_shape=jax.ShapeDtypeStruct((_MPAD, 128), _f32),
)


def kernel(x, edge_index, W1a, b1a, W1b, b1b, W2a, b2a, W2b, b2b,
           Wf1, bf1, Wf2, bf2, Ws1, bs1, Ws2, bs2,
           Wo1, bo1, Wo2, bo2, Wm1, bm1, Wm2, bm2):
    src = edge_index[0].astype(jnp.int32)
    dst = edge_index[1].astype(jnp.int32)
    pad_i = jnp.arange(_EPAD - _E, dtype=jnp.int32) % (_NP - _N)
    src_p = jnp.concatenate([src, _N + pad_i])
    dst_p = jnp.concatenate([dst, _N + (_NP - _N - 1) - pad_i])
    xp = jnp.pad(x.astype(_f32), ((0, _NP - _N), (0, 0)))
    z_nd = jnp.zeros((_NP, _HID), _f32)
    ones_ch = jnp.ones((_CH, _HID), _f32)

    (cnt,) = _count_sc(dst_p, z_nd, ones_ch)
    p1, q1 = _pre_tc(xp, W1a, b1a.reshape(1, -1))
    (acc1,) = _edge_sc(src_p, dst_p, p1, q1, z_nd)
    p2, q2 = _mid_tc(acc1, cnt, W1b, b1b.reshape(1, -1), W2a, b2a.reshape(1, -1))
    (acc2,) = _edge_sc(src_p, dst_p, p2, q2, z_nd)

    wh = jnp.concatenate([Wf1, Ws1, Wo1, Wm1], axis=1)
    bh = jnp.concatenate([bf1, bs1, bo1, bm1]).reshape(1, -1)
    wbd = jnp.zeros((4 * _HID, 128), _f32)
    wbd = wbd.at[0:128, 0:32].set(Wf2)
    wbd = wbd.at[128:256, 32:38].set(Ws2)
    wbd = wbd.at[256:384, 38:68].set(Wo2)
    wbd = wbd.at[384:512, 68:79].set(Wm2)
    bc = jnp.zeros((128,), _f32)
    bc = bc.at[0:32].set(bf2).at[32:38].set(bs2).at[38:68].set(bo2).at[68:79].set(bm2)

    out = _head_tc(acc2, cnt, W2b, b2b.reshape(1, -1), wh, bh, wbd,
                   bc.reshape(1, -1))
    return out[:_M, :79]
